# baseline (speedup 1.0000x reference)
import functools

import jax
import jax.numpy as jnp
from jax import lax
from jax.experimental import pallas as pl
from jax.experimental.pallas import tpu as pltpu
from jax.experimental.pallas import tpu_sc as plsc

VOCAB = 1000000
EMBED_DIM = 32
BATCH, N_DAYS, N_MSGS, N_WORDS = 1024, 5, 20, 20
B_TOTAL = BATCH * N_DAYS * N_MSGS * N_WORDS

NUM_CORES = 2
NUM_SUBCORES = 16
NW = NUM_CORES * NUM_SUBCORES
B_PER_W = B_TOTAL // NW
CHUNK = 640
N_CHUNKS = B_PER_W // CHUNK
NBUF = 4
LOOKAHEAD = 2
N_GROUPS = N_CHUNKS // NBUF


@jax.jit
def _sc_embed_gather(x_flat, table16):
    mesh = plsc.VectorSubcoreMesh(core_axis_name="c", subcore_axis_name="s")

    @functools.partial(
        pl.kernel,
        mesh=mesh,
        out_type=jax.ShapeDtypeStruct((B_TOTAL, EMBED_DIM), jnp.float32),
        scratch_types=[
            [pltpu.VMEM((CHUNK,), jnp.int32)] * NBUF,
            [pltpu.VMEM((CHUNK, EMBED_DIM), jnp.bfloat16)] * NBUF,
            [pltpu.VMEM((CHUNK, EMBED_DIM), jnp.float32)] * NBUF,
            [pltpu.SemaphoreType.DMA] * NBUF,
            [pltpu.SemaphoreType.DMA] * NBUF,
            [pltpu.SemaphoreType.DMA] * NBUF,
        ],
        compiler_params=pltpu.CompilerParams(
            use_tc_tiling_on_sc=False, needs_layout_passes=False),
    )
    def k(idx_hbm, tab_hbm, out_hbm, idx_v, rows16, rows32, isems, gsems, osems):
        wid = lax.axis_index("s") * NUM_CORES + lax.axis_index("c")
        base = wid * B_PER_W

        def idx_copy(c, s):
            return pltpu.make_async_copy(
                idx_hbm.at[pl.ds(base + c * CHUNK, CHUNK)], idx_v[s], isems[s])

        def gather(c, s):
            return pltpu.make_async_copy(
                tab_hbm.at[idx_v[s]], rows16[s], gsems[0])

        def out_copy(c, s):
            return pltpu.make_async_copy(
                rows32[s], out_hbm.at[pl.ds(base + c * CHUNK, CHUNK)], osems[s])

        cols2 = lax.iota(jnp.int32, 16) * 2
        mask_hi = jnp.full((16,), 0xFFFF0000, jnp.uint32)

        def convert(s):
            def row(i, carry):
                w = plsc.bitcast(rows16[s][i, :], jnp.uint32)
                lo = plsc.bitcast(jnp.left_shift(w, 16), jnp.float32)
                hi = plsc.bitcast(jnp.bitwise_and(w, mask_hi), jnp.float32)
                ri = jnp.full((16,), i, jnp.int32)
                plsc.store_scatter(rows32[s], [ri, cols2], lo)
                plsc.store_scatter(rows32[s], [ri, cols2 + 1], hi)
                return carry

            lax.fori_loop(0, CHUNK, row, 0)

        for s in range(NBUF):
            idx_copy(s, s).start()
        for c0 in range(LOOKAHEAD):
            idx_copy(c0, c0 % NBUF).wait()
            gather(c0, c0 % NBUF).start()

        def group(jj, carry):
            for s in range(NBUF):
                c = jj * NBUF + s
                gather(c, s).wait()
                pl.when(c >= NBUF)(lambda: out_copy(c - NBUF, s).wait())
                convert(s)
                out_copy(c, s).start()
                pl.when(c + NBUF < N_CHUNKS)(
                    lambda: idx_copy(c + NBUF, s).start())

                t = (s + LOOKAHEAD) % NBUF

                def launch():
                    idx_copy(c + LOOKAHEAD, t).wait()
                    gather(c + LOOKAHEAD, t).start()

                pl.when(c + LOOKAHEAD < N_CHUNKS)(launch)
            return carry

        lax.fori_loop(0, N_GROUPS, group, 0)

        for s in range(NBUF):
            out_copy(N_CHUNKS - NBUF + s, s).wait()

    return k(x_flat, table16)


def kernel(x, table):
    out = _sc_embed_gather(x.reshape(-1), table.astype(jnp.bfloat16))
    return out.reshape(x.shape + (EMBED_DIM,))


# speedup vs baseline: 1.1044x; 1.1044x over previous
import functools

import jax
import jax.numpy as jnp
from jax import lax
from jax.experimental import pallas as pl
from jax.experimental.pallas import tpu as pltpu
from jax.experimental.pallas import tpu_sc as plsc

VOCAB = 1000000
EMBED_DIM = 32
BATCH, N_DAYS, N_MSGS, N_WORDS = 1024, 5, 20, 20
B_TOTAL = BATCH * N_DAYS * N_MSGS * N_WORDS

NUM_CORES = 2
NUM_SUBCORES = 16
NW = NUM_CORES * NUM_SUBCORES
B_PER_W = B_TOTAL // NW
CHUNK = 640
N_CHUNKS = B_PER_W // CHUNK
NBUF = 5
LOOKAHEAD = 3
N_GROUPS = N_CHUNKS // NBUF


@jax.jit
def _sc_embed_gather(x_flat, table):
    mesh = plsc.VectorSubcoreMesh(core_axis_name="c", subcore_axis_name="s")

    @functools.partial(
        pl.kernel,
        mesh=mesh,
        out_type=jax.ShapeDtypeStruct((B_TOTAL, EMBED_DIM), jnp.float32),
        scratch_types=[
            [pltpu.VMEM((CHUNK,), jnp.int32)] * NBUF,
            [pltpu.VMEM((CHUNK, EMBED_DIM), jnp.float32)] * NBUF,
            [pltpu.SemaphoreType.DMA] * NBUF,
            [pltpu.SemaphoreType.DMA] * NBUF,
            [pltpu.SemaphoreType.DMA] * NBUF,
        ],
        compiler_params=pltpu.CompilerParams(use_tc_tiling_on_sc=False),
    )
    def k(idx_hbm, table_hbm, out_hbm, idx_v, rows_v, isems, gsems, osems):
        wid = lax.axis_index("s") * NUM_CORES + lax.axis_index("c")
        base = wid * B_PER_W

        def idx_copy(c, s):
            return pltpu.make_async_copy(
                idx_hbm.at[pl.ds(base + c * CHUNK, CHUNK)], idx_v[s], isems[s])

        def gather(c, s):
            return pltpu.make_async_copy(
                table_hbm.at[idx_v[s]], rows_v[s], gsems[0])

        def out_copy(c, s):
            return pltpu.make_async_copy(
                rows_v[s], out_hbm.at[pl.ds(base + c * CHUNK, CHUNK)], osems[s])

        for s in range(NBUF):
            idx_copy(s, s).start()
        for c0 in range(LOOKAHEAD):
            idx_copy(c0, c0 % NBUF).wait()
            gather(c0, c0 % NBUF).start()

        def group(jj, carry):
            for s in range(NBUF):
                c = jj * NBUF + s
                gather(c, s).wait()
                out_copy(c, s).start()
                pl.when(c + NBUF < N_CHUNKS)(
                    lambda: idx_copy(c + NBUF, s).start())

                t = (s + LOOKAHEAD) % NBUF

                def launch():
                    idx_copy(c + LOOKAHEAD, t).wait()
                    pl.when(c + LOOKAHEAD >= NBUF)(
                        lambda: out_copy(c + LOOKAHEAD - NBUF, t).wait())
                    gather(c + LOOKAHEAD, t).start()

                pl.when(c + LOOKAHEAD < N_CHUNKS)(launch)
            return carry

        lax.fori_loop(0, N_GROUPS, group, 0)

        for s in range(NBUF):
            out_copy(N_CHUNKS - NBUF + s, s).wait()

    return k(x_flat, table)


def kernel(x, table):
    out = _sc_embed_gather(x.reshape(-1), table)
    return out.reshape(x.shape + (EMBED_DIM,))
